# Initial kernel scaffold; baseline (speedup 1.0000x reference)
import jax
import jax.numpy as jnp
from jax.experimental import pallas as pl


def kernel(x, edge_index, W1, b1, W2, b2, W3, b3):
    raise NotImplementedError("write your pallas kernel here")


# speedup vs baseline: 10.5408x; 10.5408x over previous
import functools

import jax
import jax.numpy as jnp
from jax import lax
from jax.experimental import pallas as pl
from jax.experimental.pallas import tpu as pltpu
from jax.experimental.pallas import tpu_sc as plsc

N = 10000
D_IN = 128
D_H = 128
N_CLS = 10

NC = 2
NS = 16
NW = NC * NS
CHUNK = 128
STRIPE = 632
NPAD = NS * STRIPE

_mesh = plsc.VectorSubcoreMesh(
    core_axis_name="c", subcore_axis_name="s", num_cores=NC, num_subcores=NS)


def _wid():
    return lax.axis_index("c") * NS + lax.axis_index("s")


def _make_deg_kernel(cpt):
    @functools.partial(
        pl.kernel,
        out_type=jax.ShapeDtypeStruct((NC * NPAD,), jnp.float32),
        mesh=_mesh,
        scratch_types=[
            pltpu.VMEM((cpt, CHUNK), jnp.int32),
            pltpu.VMEM((CHUNK,), jnp.float32),
            pltpu.VMEM((640,), jnp.float32),
            pltpu.VMEM_SHARED((NPAD,), jnp.float32),
            pltpu.SemaphoreType.DMA,
        ],
    )
    def deg_kernel(dst_hbm, degp_out, dst_idx, ones_v, zb_v, deg_sh, sem):
        c = lax.axis_index("c")
        s = lax.axis_index("s")
        w = _wid()
        pltpu.sync_copy(dst_hbm.at[pl.ds(w * cpt, cpt)], dst_idx)
        for j in range(CHUNK // 16):
            ones_v[pl.ds(j * 16, 16)] = jnp.ones((16,), jnp.float32)
        for j in range(640 // 16):
            zb_v[pl.ds(j * 16, 16)] = jnp.zeros((16,), jnp.float32)
        pltpu.sync_copy(zb_v.at[pl.ds(0, STRIPE)],
                        deg_sh.at[pl.ds(s * STRIPE, STRIPE)])
        plsc.subcore_barrier()

        def body(i, carry):
            pltpu.sync_copy(ones_v, deg_sh.at[dst_idx.at[i]], add=True)
            return carry

        lax.fori_loop(0, cpt, body, 0)
        plsc.subcore_barrier()
        pltpu.sync_copy(deg_sh.at[pl.ds(s * STRIPE, STRIPE)],
                        zb_v.at[pl.ds(0, STRIPE)])
        pltpu.sync_copy(zb_v.at[pl.ds(0, STRIPE)],
                        degp_out.at[pl.ds(c * NPAD + s * STRIPE, STRIPE)])

    return deg_kernel


def _make_agg_kernel(cpt, do_u):
    acc_type = jax.ShapeDtypeStruct((NC, NPAD, D_H), jnp.float32)
    out_type = ([acc_type, jax.ShapeDtypeStruct((NC * NPAD,), jnp.float32)]
                if do_u else acc_type)
    scratch = [
        pltpu.VMEM((cpt, CHUNK), jnp.int32),
        pltpu.VMEM((cpt, CHUNK), jnp.int32),
        pltpu.VMEM((CHUNK, D_H), jnp.float32),
        pltpu.VMEM((CHUNK,), jnp.float32),
        pltpu.VMEM((640,), jnp.float32),
        pltpu.VMEM_SHARED((NPAD, D_H), jnp.float32),
        pltpu.VMEM_SHARED((NPAD,), jnp.float32),
        pltpu.SemaphoreType.DMA,
        pltpu.SemaphoreType.DMA,
    ]

    @functools.partial(pl.kernel, out_type=out_type, mesh=_mesh,
                       scratch_types=scratch)
    def agg_kernel(src_hbm, dst_hbm, hn_hbm, dis_hbm, z2_hbm, *rest):
        if do_u:
            acc_out, u_out = rest[0], rest[1]
            rest = rest[2:]
        else:
            acc_out = rest[0]
            rest = rest[1:]
        src_idx, dst_idx, rows_v, dvals_v, zb_v, acc_sh, u_sh, sem, sem2 = rest
        c = lax.axis_index("c")
        s = lax.axis_index("s")
        w = _wid()
        pltpu.sync_copy(src_hbm.at[pl.ds(w * cpt, cpt)], src_idx)
        pltpu.sync_copy(dst_hbm.at[pl.ds(w * cpt, cpt)], dst_idx)

        rbase = s * STRIPE

        @pl.when(c == 0)
        def _():
            pltpu.sync_copy(hn_hbm.at[pl.ds(rbase, STRIPE)],
                            acc_sh.at[pl.ds(rbase, STRIPE)])

        @pl.when(c == 1)
        def _():
            for j in range(5):
                m = min(128, STRIPE - j * 128)
                pltpu.sync_copy(z2_hbm.at[pl.ds(0, m)],
                                acc_sh.at[pl.ds(rbase + j * 128, m)])

        if do_u:
            for j in range(640 // 16):
                zb_v[pl.ds(j * 16, 16)] = jnp.zeros((16,), jnp.float32)
            pltpu.sync_copy(zb_v.at[pl.ds(0, STRIPE)],
                            u_sh.at[pl.ds(s * STRIPE, STRIPE)])
        plsc.subcore_barrier()

        def body(i, carry):
            pltpu.async_copy(hn_hbm.at[src_idx.at[i]], rows_v, sem).wait()
            pltpu.sync_copy(rows_v, acc_sh.at[dst_idx.at[i]], add=True)
            if do_u:
                pltpu.async_copy(dis_hbm.at[dst_idx.at[i]], dvals_v,
                                 sem2).wait()
                pltpu.sync_copy(dvals_v, u_sh.at[src_idx.at[i]], add=True)
            return carry

        lax.fori_loop(0, cpt, body, 0)
        plsc.subcore_barrier()
        pltpu.sync_copy(acc_sh.at[pl.ds(rbase, STRIPE)],
                        acc_out.at[c, pl.ds(rbase, STRIPE)])
        if do_u:
            pltpu.sync_copy(u_sh.at[pl.ds(s * STRIPE, STRIPE)],
                            zb_v.at[pl.ds(0, STRIPE)])
            pltpu.sync_copy(zb_v.at[pl.ds(0, STRIPE)],
                            u_out.at[pl.ds(c * NPAD + s * STRIPE, STRIPE)])

    return agg_kernel


BR = 1000
GRID = N // BR
BR1 = STRIPE
GRID1 = NPAD // BR1


def _dis_body(degp_ref, dis_ref):
    d = degp_ref[0, :] + degp_ref[1, :] + 1.0
    r = lax.rsqrt(d)
    col = lax.broadcasted_iota(jnp.int32, (1, NPAD), 1)
    dis_ref[...] = jnp.where(col < N, r, 0.0)


def _dis_call(degp):
    return pl.pallas_call(
        _dis_body,
        out_shape=jax.ShapeDtypeStruct((1, NPAD), jnp.float32),
    )(degp.reshape(1 * NC, NPAD))


def _mm1_body(x_ref, w_ref, dis_ref, out_ref):
    h = jnp.dot(x_ref[...], w_ref[...], preferred_element_type=jnp.float32)
    out_ref[...] = h * dis_ref[...]


def _mm1_call(xp, W1, dis_col):
    return pl.pallas_call(
        _mm1_body,
        grid=(GRID1,),
        in_specs=[
            pl.BlockSpec((BR1, D_IN), lambda i: (i, 0)),
            pl.BlockSpec((D_IN, D_H), lambda i: (0, 0)),
            pl.BlockSpec((BR1, 1), lambda i: (i, 0)),
        ],
        out_specs=pl.BlockSpec((BR1, D_H), lambda i: (i, 0)),
        out_shape=jax.ShapeDtypeStruct((NPAD, D_H), jnp.float32),
    )(xp, W1, dis_col)


def _mid_body(acc_ref, dis_ref, b_ref, w_ref, out_ref):
    agg = acc_ref[0] + acc_ref[1]
    h = jnp.maximum(agg * dis_ref[...] + b_ref[...], 0.0)
    hw = jnp.dot(h, w_ref[...], preferred_element_type=jnp.float32)
    out_ref[...] = hw * dis_ref[...]


def _mid_call(accp, dis_col, b1, W2):
    return pl.pallas_call(
        _mid_body,
        grid=(GRID,),
        in_specs=[
            pl.BlockSpec((NC, BR, D_H), lambda i: (0, i, 0)),
            pl.BlockSpec((BR, 1), lambda i: (i, 0)),
            pl.BlockSpec((1, D_H), lambda i: (0, 0)),
            pl.BlockSpec((D_H, D_H), lambda i: (0, 0)),
        ],
        out_specs=pl.BlockSpec((BR, D_H), lambda i: (i, 0)),
        out_shape=jax.ShapeDtypeStruct((NPAD, D_H), jnp.float32),
    )(accp, dis_col, b1.reshape(1, D_H), W2)


def _fin_body(acc_ref, dis_ref, b_ref, up_ref, w3_ref, b3_ref, out_ref,
              v_ref):
    i = pl.program_id(0)

    @pl.when(i == 0)
    def _():
        v_ref[...] = jnp.zeros_like(v_ref)

    agg = acc_ref[0] + acc_ref[1]
    dis = dis_ref[...]
    h2 = jnp.maximum(agg * dis + b_ref[...], 0.0)
    s = dis * (up_ref[0] + up_ref[1] + dis)
    v_ref[...] += jnp.sum(h2 * s, axis=0, keepdims=True)

    @pl.when(i == GRID - 1)
    def _():
        v = v_ref[...] * (1.0 / N)
        logits = jnp.dot(v, w3_ref[...],
                         preferred_element_type=jnp.float32) + b3_ref[...]
        m = jnp.max(logits, axis=1, keepdims=True)
        e = jnp.exp(logits - m)
        lse = jnp.log(jnp.sum(e, axis=1, keepdims=True)) + m
        out_ref[...] = logits - lse


def _fin_call(accp, dis_col, b2, up, W3, b3):
    return pl.pallas_call(
        _fin_body,
        grid=(GRID,),
        in_specs=[
            pl.BlockSpec((NC, BR, D_H), lambda i: (0, i, 0)),
            pl.BlockSpec((BR, 1), lambda i: (i, 0)),
            pl.BlockSpec((1, D_H), lambda i: (0, 0)),
            pl.BlockSpec((NC, BR, 1), lambda i: (0, i, 0)),
            pl.BlockSpec((D_H, N_CLS), lambda i: (0, 0)),
            pl.BlockSpec((1, N_CLS), lambda i: (0, 0)),
        ],
        out_specs=pl.BlockSpec((1, N_CLS), lambda i: (0, 0)),
        out_shape=jax.ShapeDtypeStruct((1, N_CLS), jnp.float32),
        scratch_shapes=[pltpu.VMEM((1, D_H), jnp.float32)],
    )(accp, dis_col, b2.reshape(1, D_H), up, W3, b3.reshape(1, N_CLS))


def kernel(x, edge_index, W1, b1, W2, b2, W3, b3):
    E = edge_index.shape[1]
    cpt = -(-E // (CHUNK * NW))
    cpt = -(-cpt // 8) * 8
    nchunks = cpt * NW
    epad = nchunks * CHUNK - E

    src = jnp.concatenate(
        [edge_index[0], jnp.zeros((epad,), jnp.int32)]).reshape(nchunks, CHUNK)
    dst = jnp.concatenate(
        [edge_index[1], jnp.full((epad,), N, jnp.int32)]).reshape(nchunks,
                                                                  CHUNK)
    z2 = jnp.zeros((CHUNK, D_H), jnp.float32)
    xp = jnp.concatenate(
        [x, jnp.zeros((NPAD - N, D_IN), jnp.float32)], axis=0)

    degp = _make_deg_kernel(cpt)(dst).reshape(NC, NPAD)
    dis_row = _dis_call(degp)
    dis_flat = dis_row.reshape(NPAD)
    dis_col = dis_row.reshape(NPAD, 1)

    hn1 = _mm1_call(xp, W1, dis_col)
    acc1, up = _make_agg_kernel(cpt, True)(src, dst, hn1, dis_flat, z2)
    hn2 = _mid_call(acc1, dis_col, b1, W2)
    acc2 = _make_agg_kernel(cpt, False)(src, dst, hn2, dis_flat, z2)
    up_col = up.reshape(NC, NPAD, 1)
    return _fin_call(acc2, dis_col, b2, up_col, W3, b3)


# speedup vs baseline: 11.8915x; 1.1281x over previous
import functools

import jax
import jax.numpy as jnp
from jax import lax
from jax.experimental import pallas as pl
from jax.experimental.pallas import tpu as pltpu
from jax.experimental.pallas import tpu_sc as plsc

N = 10000
D_IN = 128
D_H = 128
N_CLS = 10

NC = 2
NS = 16
NW = NC * NS
CHUNK = 128
GB = 2
STRIPE = 632
NPAD = NS * STRIPE

_mesh = plsc.VectorSubcoreMesh(
    core_axis_name="c", subcore_axis_name="s", num_cores=NC, num_subcores=NS)


def _wid():
    return lax.axis_index("s") * NC + lax.axis_index("c")


def _make_deg_kernel(cpt):
    @functools.partial(
        pl.kernel,
        out_type=jax.ShapeDtypeStruct((NC * NPAD,), jnp.float32),
        mesh=_mesh,
        scratch_types=[
            pltpu.VMEM((cpt, CHUNK), jnp.int32),
            pltpu.VMEM((CHUNK,), jnp.float32),
            pltpu.VMEM((640,), jnp.float32),
            pltpu.VMEM_SHARED((NPAD,), jnp.float32),
            pltpu.SemaphoreType.DMA,
        ],
    )
    def deg_kernel(dst_hbm, degp_out, dst_idx, ones_v, zb_v, deg_sh, sem):
        c = lax.axis_index("c")
        s = lax.axis_index("s")
        w = _wid()
        pltpu.sync_copy(dst_hbm.at[pl.ds(w * cpt, cpt)], dst_idx)
        for j in range(CHUNK // 16):
            ones_v[pl.ds(j * 16, 16)] = jnp.ones((16,), jnp.float32)
        for j in range(640 // 16):
            zb_v[pl.ds(j * 16, 16)] = jnp.zeros((16,), jnp.float32)
        pltpu.sync_copy(zb_v.at[pl.ds(0, STRIPE)],
                        deg_sh.at[pl.ds(s * STRIPE, STRIPE)])
        plsc.subcore_barrier()

        def body(i, carry):
            pltpu.sync_copy(ones_v, deg_sh.at[dst_idx.at[i]], add=True)
            return carry

        lax.fori_loop(0, cpt, body, 0)
        plsc.subcore_barrier()
        pltpu.sync_copy(deg_sh.at[pl.ds(s * STRIPE, STRIPE)],
                        zb_v.at[pl.ds(0, STRIPE)])
        pltpu.sync_copy(zb_v.at[pl.ds(0, STRIPE)],
                        degp_out.at[pl.ds(c * NPAD + s * STRIPE, STRIPE)])

    return deg_kernel


def _make_agg_kernel(cpt, do_u):
    acc_type = jax.ShapeDtypeStruct((NC, NPAD, D_H), jnp.float32)
    out_type = ([acc_type, jax.ShapeDtypeStruct((NC * NPAD,), jnp.float32)]
                if do_u else acc_type)
    half = cpt // 2
    scratch = [
        pltpu.VMEM((half, CHUNK), jnp.int32),
        pltpu.VMEM((half, CHUNK), jnp.int32),
        pltpu.VMEM((GB * CHUNK, D_H), jnp.float32),
        pltpu.VMEM_SHARED((NPAD, D_H), jnp.float32),
        pltpu.SemaphoreType.DMA,
        pltpu.SemaphoreType.DMA,
        pltpu.VMEM((640,), jnp.float32),
    ]
    if do_u:
        scratch += [
            pltpu.VMEM((GB * CHUNK,), jnp.float32),
            pltpu.VMEM_SHARED((NPAD,), jnp.float32),
        ]

    @functools.partial(pl.kernel, out_type=out_type, mesh=_mesh,
                       scratch_types=scratch)
    def agg_kernel(src_hbm, dst_hbm, hn_hbm, dis_hbm, z2_hbm, *rest):
        if do_u:
            acc_out, u_out = rest[0], rest[1]
            (src_idx, dst_idx, rows_v, acc_sh, sem, sem2, zb_v,
             dvals_v, u_sh) = rest[2:]
        else:
            acc_out = rest[0]
            src_idx, dst_idx, rows_v, acc_sh, sem, sem2, zb_v = rest[1:]
        c = lax.axis_index("c")
        s = lax.axis_index("s")
        w = _wid()

        rbase = s * STRIPE

        @pl.when(c == 0)
        def _():
            pltpu.sync_copy(hn_hbm.at[pl.ds(rbase, STRIPE)],
                            acc_sh.at[pl.ds(rbase, STRIPE)])

        @pl.when(c == 1)
        def _():
            for j in range(5):
                m = min(128, STRIPE - j * 128)
                pltpu.sync_copy(z2_hbm.at[pl.ds(0, m)],
                                acc_sh.at[pl.ds(rbase + j * 128, m)])

        if do_u:
            for j in range(640 // 16):
                zb_v[pl.ds(j * 16, 16)] = jnp.zeros((16,), jnp.float32)
            pltpu.sync_copy(zb_v.at[pl.ds(0, STRIPE)],
                            u_sh.at[pl.ds(s * STRIPE, STRIPE)])
        plsc.subcore_barrier()

        def half_body(h, carry):
            pltpu.sync_copy(src_hbm.at[pl.ds(w * cpt + h * half, half)],
                            src_idx)
            pltpu.sync_copy(dst_hbm.at[pl.ds(w * cpt + h * half, half)],
                            dst_idx)

            def body(g, carry2):
                base = g * GB
                gds = []
                for j in range(GB):
                    gds.append(pltpu.async_copy(
                        hn_hbm.at[src_idx.at[base + j]],
                        rows_v.at[pl.ds(j * CHUNK, CHUNK)], sem))
                if do_u:
                    for j in range(GB):
                        gds.append(pltpu.async_copy(
                            dis_hbm.at[dst_idx.at[base + j]],
                            dvals_v.at[pl.ds(j * CHUNK, CHUNK)], sem))
                sds = []
                for j in range(GB):
                    gds[j].wait()
                    sds.append(pltpu.async_copy(
                        rows_v.at[pl.ds(j * CHUNK, CHUNK)],
                        acc_sh.at[dst_idx.at[base + j]], sem2, add=True))
                if do_u:
                    for j in range(GB):
                        gds[GB + j].wait()
                        sds.append(pltpu.async_copy(
                            dvals_v.at[pl.ds(j * CHUNK, CHUNK)],
                            u_sh.at[src_idx.at[base + j]], sem2, add=True))
                for d in sds:
                    d.wait()
                return carry2

            lax.fori_loop(0, half // GB, body, 0)
            return carry

        lax.fori_loop(0, 2, half_body, 0)
        plsc.subcore_barrier()
        pltpu.sync_copy(acc_sh.at[pl.ds(rbase, STRIPE)],
                        acc_out.at[c, pl.ds(rbase, STRIPE)])
        if do_u:
            pltpu.sync_copy(u_sh.at[pl.ds(s * STRIPE, STRIPE)],
                            zb_v.at[pl.ds(0, STRIPE)])
            pltpu.sync_copy(zb_v.at[pl.ds(0, STRIPE)],
                            u_out.at[pl.ds(c * NPAD + s * STRIPE, STRIPE)])

    return agg_kernel


BR = 1000
GRID = N // BR
BR1 = STRIPE
GRID1 = NPAD // BR1


def _dis_body(degp_ref, dis_ref):
    d = degp_ref[0, :] + degp_ref[1, :] + 1.0
    r = lax.rsqrt(d)
    col = lax.broadcasted_iota(jnp.int32, (1, NPAD), 1)
    dis_ref[...] = jnp.where(col < N, r, 0.0)


def _dis_call(degp):
    return pl.pallas_call(
        _dis_body,
        out_shape=jax.ShapeDtypeStruct((1, NPAD), jnp.float32),
    )(degp.reshape(1 * NC, NPAD))


def _mm1_body(x_ref, w_ref, dis_ref, out_ref):
    h = jnp.dot(x_ref[...], w_ref[...], preferred_element_type=jnp.float32)
    out_ref[...] = h * dis_ref[...]


def _mm1_call(xp, W1, dis_col):
    return pl.pallas_call(
        _mm1_body,
        grid=(GRID1,),
        in_specs=[
            pl.BlockSpec((BR1, D_IN), lambda i: (i, 0)),
            pl.BlockSpec((D_IN, D_H), lambda i: (0, 0)),
            pl.BlockSpec((BR1, 1), lambda i: (i, 0)),
        ],
        out_specs=pl.BlockSpec((BR1, D_H), lambda i: (i, 0)),
        out_shape=jax.ShapeDtypeStruct((NPAD, D_H), jnp.float32),
    )(xp, W1, dis_col)


def _mid_body(acc_ref, dis_ref, b_ref, w_ref, out_ref):
    agg = acc_ref[0] + acc_ref[1]
    h = jnp.maximum(agg * dis_ref[...] + b_ref[...], 0.0)
    hw = jnp.dot(h, w_ref[...], preferred_element_type=jnp.float32)
    out_ref[...] = hw * dis_ref[...]


def _mid_call(accp, dis_col, b1, W2):
    return pl.pallas_call(
        _mid_body,
        grid=(GRID,),
        in_specs=[
            pl.BlockSpec((NC, BR, D_H), lambda i: (0, i, 0)),
            pl.BlockSpec((BR, 1), lambda i: (i, 0)),
            pl.BlockSpec((1, D_H), lambda i: (0, 0)),
            pl.BlockSpec((D_H, D_H), lambda i: (0, 0)),
        ],
        out_specs=pl.BlockSpec((BR, D_H), lambda i: (i, 0)),
        out_shape=jax.ShapeDtypeStruct((NPAD, D_H), jnp.float32),
    )(accp, dis_col, b1.reshape(1, D_H), W2)


def _fin_body(acc_ref, dis_ref, b_ref, up_ref, w3_ref, b3_ref, out_ref,
              v_ref):
    i = pl.program_id(0)

    @pl.when(i == 0)
    def _():
        v_ref[...] = jnp.zeros_like(v_ref)

    agg = acc_ref[0] + acc_ref[1]
    dis = dis_ref[...]
    h2 = jnp.maximum(agg * dis + b_ref[...], 0.0)
    s = dis * (up_ref[0] + up_ref[1] + dis)
    v_ref[...] += jnp.sum(h2 * s, axis=0, keepdims=True)

    @pl.when(i == GRID - 1)
    def _():
        v = v_ref[...] * (1.0 / N)
        logits = jnp.dot(v, w3_ref[...],
                         preferred_element_type=jnp.float32) + b3_ref[...]
        m = jnp.max(logits, axis=1, keepdims=True)
        e = jnp.exp(logits - m)
        lse = jnp.log(jnp.sum(e, axis=1, keepdims=True)) + m
        out_ref[...] = logits - lse


def _fin_call(accp, dis_col, b2, up, W3, b3):
    return pl.pallas_call(
        _fin_body,
        grid=(GRID,),
        in_specs=[
            pl.BlockSpec((NC, BR, D_H), lambda i: (0, i, 0)),
            pl.BlockSpec((BR, 1), lambda i: (i, 0)),
            pl.BlockSpec((1, D_H), lambda i: (0, 0)),
            pl.BlockSpec((NC, BR, 1), lambda i: (0, i, 0)),
            pl.BlockSpec((D_H, N_CLS), lambda i: (0, 0)),
            pl.BlockSpec((1, N_CLS), lambda i: (0, 0)),
        ],
        out_specs=pl.BlockSpec((1, N_CLS), lambda i: (0, 0)),
        out_shape=jax.ShapeDtypeStruct((1, N_CLS), jnp.float32),
        scratch_shapes=[pltpu.VMEM((1, D_H), jnp.float32)],
    )(accp, dis_col, b2.reshape(1, D_H), up, W3, b3.reshape(1, N_CLS))


def kernel(x, edge_index, W1, b1, W2, b2, W3, b3):
    E = edge_index.shape[1]
    cpt = -(-E // (CHUNK * NW))
    cpt = -(-cpt // 8) * 8
    nchunks = cpt * NW
    epad = nchunks * CHUNK - E

    src = jnp.concatenate(
        [edge_index[0], jnp.zeros((epad,), jnp.int32)]).reshape(nchunks, CHUNK)
    dst = jnp.concatenate(
        [edge_index[1], jnp.full((epad,), N, jnp.int32)]).reshape(nchunks,
                                                                  CHUNK)
    z2 = jnp.zeros((CHUNK, D_H), jnp.float32)
    xp = jnp.concatenate(
        [x, jnp.zeros((NPAD - N, D_IN), jnp.float32)], axis=0)

    degp = _make_deg_kernel(cpt)(dst).reshape(NC, NPAD)
    dis_row = _dis_call(degp)
    dis_flat = dis_row.reshape(NPAD)
    dis_col = dis_row.reshape(NPAD, 1)

    hn1 = _mm1_call(xp, W1, dis_col)
    acc1, up = _make_agg_kernel(cpt, True)(src, dst, hn1, dis_flat, z2)
    hn2 = _mid_call(acc1, dis_col, b1, W2)
    acc2 = _make_agg_kernel(cpt, False)(src, dst, hn2, dis_flat, z2)
    up_col = up.reshape(NC, NPAD, 1)
    return _fin_call(acc2, dis_col, b2, up_col, W3, b3)
